# baseline (speedup 1.0000x reference)
import jax
import jax.numpy as jnp
from jax.experimental import pallas as pl
from jax.experimental.pallas import tpu as pltpu

N = 10000
D = 128
C = 2
BM1 = 400
BM2 = 400


def _matvec_block(a_blk, yt_ref):
    cols = [jnp.sum(a_blk * yt_ref[c:c + 1, :], axis=1, keepdims=True)
            for c in range(C)]
    return jnp.concatenate(cols, axis=1)


def _sweep1_body(x0_ref, w_ref, adj_ref, y0t_ref, y1_ref):
    @pl.when(pl.program_id(0) == 0)
    def _init():
        y0t_ref[...] = jax.lax.dot_general(
            w_ref[...], x0_ref[...], (((1,), (1,)), ((), ())),
            preferred_element_type=jnp.float32)

    y1_ref[...] = _matvec_block(adj_ref[...], y0t_ref)


def _sweep2_body(y1t_ref, adj_ref, y2_ref):
    y2_ref[...] = _matvec_block(adj_ref[...], y1t_ref)


def _sweep3_body(a_ref, b_ref, y2t_ref, y0_ref, y1_ref, y2_ref, adj_ref,
                 out_ref):
    y3_blk = _matvec_block(adj_ref[...], y2t_ref)
    out_ref[...] = (a_ref[0] * y0_ref[...] + a_ref[1] * y1_ref[...]
                    + a_ref[2] * y2_ref[...] + a_ref[3] * y3_blk
                    + b_ref[...])


def kernel(node_indices, adj_norm, embedding, W, b, alpha):
    a = jax.nn.softmax(alpha.astype(jnp.float32), axis=0)
    x0 = jnp.take(embedding, node_indices, axis=0)
    b2 = b.reshape(1, C)

    y0t, y1 = pl.pallas_call(
        _sweep1_body,
        grid=(N // BM1,),
        in_specs=[
            pl.BlockSpec((N, D), lambda i: (0, 0)),
            pl.BlockSpec((C, D), lambda i: (0, 0)),
            pl.BlockSpec((BM1, N), lambda i: (i, 0)),
        ],
        out_specs=[
            pl.BlockSpec((C, N), lambda i: (0, 0)),
            pl.BlockSpec((BM1, C), lambda i: (i, 0)),
        ],
        out_shape=[
            jax.ShapeDtypeStruct((C, N), jnp.float32),
            jax.ShapeDtypeStruct((N, C), jnp.float32),
        ],
        compiler_params=pltpu.CompilerParams(
            dimension_semantics=("arbitrary",),
        ),
    )(x0, W, adj_norm)

    y0 = y0t.T
    y1t = y1.T

    grid2 = (pl.cdiv(N, BM2),)
    y2 = pl.pallas_call(
        _sweep2_body,
        grid=grid2,
        in_specs=[
            pl.BlockSpec((C, N), lambda i: (0, 0)),
            pl.BlockSpec((BM2, N), lambda i: (i, 0)),
        ],
        out_specs=pl.BlockSpec((BM2, C), lambda i: (i, 0)),
        out_shape=jax.ShapeDtypeStruct((N, C), jnp.float32),
        compiler_params=pltpu.CompilerParams(
            dimension_semantics=("arbitrary",),
        ),
    )(y1t, adj_norm)

    y2t = y2.T

    out = pl.pallas_call(
        _sweep3_body,
        grid=grid2,
        in_specs=[
            pl.BlockSpec(memory_space=pltpu.SMEM),
            pl.BlockSpec((1, C), lambda i: (0, 0)),
            pl.BlockSpec((C, N), lambda i: (0, 0)),
            pl.BlockSpec((BM2, C), lambda i: (i, 0)),
            pl.BlockSpec((BM2, C), lambda i: (i, 0)),
            pl.BlockSpec((BM2, C), lambda i: (i, 0)),
            pl.BlockSpec((BM2, N), lambda i: (i, 0)),
        ],
        out_specs=pl.BlockSpec((BM2, C), lambda i: (i, 0)),
        out_shape=jax.ShapeDtypeStruct((N, C), jnp.float32),
        compiler_params=pltpu.CompilerParams(
            dimension_semantics=("arbitrary",),
        ),
    )(a, b2, y2t, y0, y1, y2, adj_norm)
    return out


# speedup vs baseline: 1.1365x; 1.1365x over previous
import jax
import jax.numpy as jnp
from jax.experimental import pallas as pl
from jax.experimental.pallas import tpu as pltpu

N = 10000
D = 128
C = 2
BM1 = 400
BM2 = 800


def _sweep1_body(x0_ref, adj_ref, x1_ref, a16_ref, xb_ref):
    @pl.when(pl.program_id(0) == 0)
    def _init():
        xb_ref[...] = x0_ref[...].astype(jnp.bfloat16)

    a16_blk = adj_ref[...].astype(jnp.bfloat16)
    x1_ref[...] = jnp.dot(a16_blk, xb_ref[...],
                          preferred_element_type=jnp.float32)
    a16_ref[...] = a16_blk


def _sweep2_body(x1_ref, a16_ref, x2_ref, xb_ref):
    @pl.when(pl.program_id(0) == 0)
    def _init():
        xb_ref[...] = x1_ref[...].astype(jnp.bfloat16)

    x2_ref[...] = jnp.dot(a16_ref[...], xb_ref[...],
                          preferred_element_type=jnp.float32)


def _sweep3_body(a_ref, b_ref, wb_ref, x2f_ref, x0_ref, x1_ref, x2_ref,
                 a16_ref, out_ref, xb_ref):
    @pl.when(pl.program_id(0) == 0)
    def _init():
        xb_ref[...] = x2f_ref[...].astype(jnp.bfloat16)

    x3_blk = jnp.dot(a16_ref[...], xb_ref[...],
                     preferred_element_type=jnp.float32)
    xf = (a_ref[0] * x0_ref[...] + a_ref[1] * x1_ref[...]
          + a_ref[2] * x2_ref[...] + a_ref[3] * x3_blk)
    out_ref[...] = jnp.dot(xf.astype(jnp.bfloat16), wb_ref[...],
                           preferred_element_type=jnp.float32) + b_ref[...]


def kernel(node_indices, adj_norm, embedding, W, b, alpha):
    a = jax.nn.softmax(alpha.astype(jnp.float32), axis=0)
    x0 = jnp.take(embedding, node_indices, axis=0)
    b2 = b.reshape(1, C)
    wb = W.T.astype(jnp.bfloat16)

    x1, a16 = pl.pallas_call(
        _sweep1_body,
        grid=(N // BM1,),
        in_specs=[
            pl.BlockSpec((N, D), lambda i: (0, 0)),
            pl.BlockSpec((BM1, N), lambda i: (i, 0)),
        ],
        out_specs=[
            pl.BlockSpec((BM1, D), lambda i: (i, 0)),
            pl.BlockSpec((BM1, N), lambda i: (i, 0)),
        ],
        out_shape=[
            jax.ShapeDtypeStruct((N, D), jnp.float32),
            jax.ShapeDtypeStruct((N, N), jnp.bfloat16),
        ],
        scratch_shapes=[pltpu.VMEM((N, D), jnp.bfloat16)],
        compiler_params=pltpu.CompilerParams(
            dimension_semantics=("arbitrary",),
        ),
    )(x0, adj_norm)

    grid2 = (pl.cdiv(N, BM2),)
    x2 = pl.pallas_call(
        _sweep2_body,
        grid=grid2,
        in_specs=[
            pl.BlockSpec((N, D), lambda i: (0, 0)),
            pl.BlockSpec((BM2, N), lambda i: (i, 0)),
        ],
        out_specs=pl.BlockSpec((BM2, D), lambda i: (i, 0)),
        out_shape=jax.ShapeDtypeStruct((N, D), jnp.float32),
        scratch_shapes=[pltpu.VMEM((N, D), jnp.bfloat16)],
        compiler_params=pltpu.CompilerParams(
            dimension_semantics=("arbitrary",),
        ),
    )(x1, a16)

    out = pl.pallas_call(
        _sweep3_body,
        grid=grid2,
        in_specs=[
            pl.BlockSpec(memory_space=pltpu.SMEM),
            pl.BlockSpec((1, C), lambda i: (0, 0)),
            pl.BlockSpec((D, C), lambda i: (0, 0)),
            pl.BlockSpec((N, D), lambda i: (0, 0)),
            pl.BlockSpec((BM2, D), lambda i: (i, 0)),
            pl.BlockSpec((BM2, D), lambda i: (i, 0)),
            pl.BlockSpec((BM2, D), lambda i: (i, 0)),
            pl.BlockSpec((BM2, N), lambda i: (i, 0)),
        ],
        out_specs=pl.BlockSpec((BM2, C), lambda i: (i, 0)),
        out_shape=jax.ShapeDtypeStruct((N, C), jnp.float32),
        scratch_shapes=[pltpu.VMEM((N, D), jnp.bfloat16)],
        compiler_params=pltpu.CompilerParams(
            dimension_semantics=("arbitrary",),
        ),
    )(a, b2, wb, x2, x0, x1, x2, a16)
    return out
